# Initial kernel scaffold; baseline (speedup 1.0000x reference)
import jax
import jax.numpy as jnp
from jax.experimental import pallas as pl


def kernel(x, edge_attr, enn_w1, enn_b1, enn_w2, enn_b2, root_w, nn_bias, conv_ws, conv_bs, mlp_ws, mlp_bs, edge_index):
    raise NotImplementedError("write your pallas kernel here")


# speedup vs baseline: 5.8256x; 5.8256x over previous
import functools

import jax
import jax.numpy as jnp
from jax import lax
from jax.experimental import pallas as pl
from jax.experimental.pallas import tpu as pltpu
from jax.experimental.pallas import tpu_sc as plsc

N = 10000
NP = 10240
E = 320000
INC = 13
INP = 16
EMB = 32
HID = 128
NCONV = 8
MW = 48

NC = 2
NS = 16
K = 80

ES_PER_W = E // (NC * NS)
ES_CHUNKS = ES_PER_W // K
FS_PER_S = E // NS
FS_G = 25
FS_GC = 10
ROWS_PER_S = NP // NS

RB = 1024
EB = 2000

F32 = jnp.float32


def _mesh():
    return plsc.VectorSubcoreMesh(core_axis_name="c", subcore_axis_name="s")


@functools.partial(
    pl.kernel,
    compiler_params=pltpu.CompilerParams(use_tc_tiling_on_sc=False),
    out_type=jax.ShapeDtypeStruct((E, INP), F32),
    mesh=_mesh(),
    scratch_types=[
        pltpu.VMEM_SHARED((NP, INP), F32),
        pltpu.VMEM((ES_CHUNKS, K), jnp.int32),
        pltpu.VMEM((K, INP), F32),
        pltpu.SemaphoreType.DMA,
    ],
)
def _sc_gather_x(xp_hbm, srci_hbm, out_hbm, table_sp, idx_v, buf, sem):
    c = lax.axis_index("c")
    s = lax.axis_index("s")
    wid = c * NS + s
    rs = pl.ds(s * ROWS_PER_S, ROWS_PER_S)
    pltpu.sync_copy(xp_hbm.at[rs], table_sp.at[rs])
    pltpu.sync_copy(srci_hbm.at[wid], idx_v)
    plsc.subcore_barrier()
    base = wid * ES_PER_W

    def body(j, carry):
        pltpu.async_copy(table_sp.at[idx_v.at[j]], buf, sem).wait()
        pltpu.sync_copy(buf, out_hbm.at[pl.ds(base + j * K, K)])
        return carry

    lax.fori_loop(0, ES_CHUNKS, body, 0)


@functools.partial(
    pl.kernel,
    compiler_params=pltpu.CompilerParams(use_tc_tiling_on_sc=False),
    out_type=jax.ShapeDtypeStruct((NC, NP, MW), F32),
    mesh=_mesh(),
    scratch_types=[
        pltpu.VMEM_SHARED((NP, MW), F32),
        pltpu.VMEM((ES_CHUNKS, K), jnp.int32),
        pltpu.VMEM((K, MW), F32),
        pltpu.SemaphoreType.DMA,
    ],
)
def _sc_scatter_msgs(msgs_hbm, dsti_hbm, zeros_hbm, out_hbm, accum_sp, idx_v,
                     buf, sem):
    c = lax.axis_index("c")
    s = lax.axis_index("s")
    wid = c * NS + s
    rs = pl.ds(s * ROWS_PER_S, ROWS_PER_S)
    pltpu.sync_copy(zeros_hbm.at[rs], accum_sp.at[rs])
    pltpu.sync_copy(dsti_hbm.at[wid], idx_v)
    plsc.subcore_barrier()
    base = wid * ES_PER_W

    def body(j, carry):
        pltpu.async_copy(msgs_hbm.at[pl.ds(base + j * K, K)], buf, sem).wait()
        pltpu.sync_copy(buf, accum_sp.at[idx_v.at[j]], add=True)
        return carry

    lax.fori_loop(0, ES_CHUNKS, body, 0)
    plsc.subcore_barrier()
    pltpu.sync_copy(accum_sp.at[rs], out_hbm.at[c, rs])


@functools.partial(
    pl.kernel,
    compiler_params=pltpu.CompilerParams(use_tc_tiling_on_sc=False),
    out_type=jax.ShapeDtypeStruct((NC, NP, 64), F32),
    mesh=_mesh(),
    scratch_types=[
        pltpu.VMEM_SHARED((NP, 64), F32),
        pltpu.VMEM_SHARED((NP, 64), F32),
        pltpu.VMEM((FS_GC, K), jnp.int32),
        pltpu.VMEM((FS_GC, K), jnp.int32),
        pltpu.VMEM((K, 64), F32),
        pltpu.SemaphoreType.DMA,
    ],
)
def _sc_gcn_agg(hp_hbm, srci_hbm, dsti_hbm, zeros_hbm, out_hbm, table_sp,
                accum_sp, src_blk, dst_blk, buf, sem):
    c = lax.axis_index("c")
    s = lax.axis_index("s")
    rs = pl.ds(s * ROWS_PER_S, ROWS_PER_S)
    pltpu.sync_copy(hp_hbm.at[c, rs], table_sp.at[rs])
    pltpu.sync_copy(zeros_hbm.at[rs], accum_sp.at[rs])
    plsc.subcore_barrier()

    def group(g, carry):
        pltpu.sync_copy(srci_hbm.at[s, g], src_blk)
        pltpu.sync_copy(dsti_hbm.at[s, g], dst_blk)

        def body(jj, carry2):
            pltpu.async_copy(table_sp.at[src_blk.at[jj]], buf, sem).wait()
            pltpu.sync_copy(buf, accum_sp.at[dst_blk.at[jj]], add=True)
            return carry2

        return lax.fori_loop(0, FS_GC, body, carry)

    lax.fori_loop(0, FS_G, group, 0)
    plsc.subcore_barrier()
    pltpu.sync_copy(accum_sp.at[rs], out_hbm.at[c, rs])


@functools.partial(
    pl.kernel,
    compiler_params=pltpu.CompilerParams(use_tc_tiling_on_sc=False),
    out_type=[
        jax.ShapeDtypeStruct((NC, E, 64), F32),
        jax.ShapeDtypeStruct((NC, E, 64), F32),
    ],
    mesh=_mesh(),
    scratch_types=[
        pltpu.VMEM_SHARED((NP, 64), F32),
        pltpu.VMEM_SHARED((NP, 64), F32),
        pltpu.VMEM((FS_GC, K), jnp.int32),
        pltpu.VMEM((FS_GC, K), jnp.int32),
        pltpu.VMEM((K, 64), F32),
        pltpu.VMEM((K, 64), F32),
        pltpu.SemaphoreType.DMA,
        pltpu.SemaphoreType.DMA,
    ],
)
def _sc_head_gather(a_hbm, b_hbm, srci_hbm, dsti_hbm, as_hbm, bs_hbm, ta_sp,
                    tb_sp, src_blk, dst_blk, bufa, bufb, sema, semb):
    c = lax.axis_index("c")
    s = lax.axis_index("s")
    rs = pl.ds(s * ROWS_PER_S, ROWS_PER_S)
    pltpu.sync_copy(a_hbm.at[c, rs], ta_sp.at[rs])
    pltpu.sync_copy(b_hbm.at[c, rs], tb_sp.at[rs])
    plsc.subcore_barrier()
    base = s * FS_PER_S

    def group(g, carry):
        pltpu.sync_copy(srci_hbm.at[s, g], src_blk)
        pltpu.sync_copy(dsti_hbm.at[s, g], dst_blk)

        def body(jj, carry2):
            ca = pltpu.async_copy(ta_sp.at[src_blk.at[jj]], bufa, sema)
            cb = pltpu.async_copy(tb_sp.at[dst_blk.at[jj]], bufb, semb)
            ca.wait()
            cb.wait()
            off = base + (g * FS_GC + jj) * K
            pltpu.sync_copy(bufa, as_hbm.at[c, pl.ds(off, K)])
            pltpu.sync_copy(bufb, bs_hbm.at[c, pl.ds(off, K)])
            return carry2

        return lax.fori_loop(0, FS_GC, body, carry)

    lax.fori_loop(0, FS_G, group, 0)


def _dot(a, b):
    return jnp.dot(a, b, preferred_element_type=F32)


def _tc_msgs_body(ea_ref, xs_ref, w1_ref, b1_ref, w2all_ref, out_ref):
    h = jnp.maximum(_dot(ea_ref[...], w1_ref[...]) + b1_ref[...], 0.0)
    z = _dot(xs_ref[...], w2all_ref[...])
    m = z[:, 8 * EMB:9 * EMB]
    for k in range(8):
        m = m + h[:, k:k + 1] * z[:, k * EMB:(k + 1) * EMB]
    out_ref[:, 0:EMB] = m
    out_ref[:, EMB:EMB + 1] = jnp.ones((EB, 1), F32)
    out_ref[:, EMB + 1:MW] = jnp.zeros((EB, MW - EMB - 1), F32)


def _tc_msgs(edge_attr, xs, enn_w1, enn_b1, w2all):
    return pl.pallas_call(
        _tc_msgs_body,
        grid=(E // EB,),
        in_specs=[
            pl.BlockSpec((EB, 3), lambda i: (i, 0)),
            pl.BlockSpec((EB, INP), lambda i: (i, 0)),
            pl.BlockSpec((3, 8), lambda i: (0, 0)),
            pl.BlockSpec((1, 8), lambda i: (0, 0)),
            pl.BlockSpec((INP, 9 * EMB), lambda i: (0, 0)),
        ],
        out_specs=pl.BlockSpec((EB, MW), lambda i: (i, 0)),
        out_shape=jax.ShapeDtypeStruct((E, MW), F32),
    )(edge_attr, xs, enn_w1, enn_b1, w2all)


def _tc_x1_body(ad_ref, xp_ref, rw_ref, nb_ref, x1_ref, dinv_ref):
    ad = ad_ref[0] + ad_ref[1]
    agg = ad[:, :EMB]
    deg = ad[:, EMB:EMB + 1] + 1.0
    dinv_ref[...] = lax.rsqrt(deg)
    x1_ref[...] = jnp.maximum(
        agg + _dot(xp_ref[...], rw_ref[...]) + nb_ref[...], 0.0)


def _tc_x1(aggdeg, xp, root_w, nn_bias):
    return pl.pallas_call(
        _tc_x1_body,
        grid=(NP // RB,),
        in_specs=[
            pl.BlockSpec((NC, RB, MW), lambda i: (0, i, 0)),
            pl.BlockSpec((RB, INP), lambda i: (i, 0)),
            pl.BlockSpec((INP, EMB), lambda i: (0, 0)),
            pl.BlockSpec((1, EMB), lambda i: (0, 0)),
        ],
        out_specs=[
            pl.BlockSpec((RB, EMB), lambda i: (i, 0)),
            pl.BlockSpec((RB, 1), lambda i: (i, 0)),
        ],
        out_shape=[
            jax.ShapeDtypeStruct((NP, EMB), F32),
            jax.ShapeDtypeStruct((NP, 1), F32),
        ],
    )(aggdeg, xp, root_w, nn_bias)


def _tc_gcn_pre_body(x_ref, w_ref, dinv_ref, hp_ref):
    hp = dinv_ref[...] * _dot(x_ref[...], w_ref[...])
    hp_ref[0] = hp[:, :64]
    hp_ref[1] = hp[:, 64:]


def _tc_gcn_pre(xc, w, dinv):
    cin = xc.shape[1]
    return pl.pallas_call(
        _tc_gcn_pre_body,
        grid=(NP // RB,),
        in_specs=[
            pl.BlockSpec((RB, cin), lambda i: (i, 0)),
            pl.BlockSpec((cin, HID), lambda i: (0, 0)),
            pl.BlockSpec((RB, 1), lambda i: (i, 0)),
        ],
        out_specs=pl.BlockSpec((NC, RB, 64), lambda i: (0, i, 0)),
        out_shape=jax.ShapeDtypeStruct((NC, NP, 64), F32),
    )(xc, w, dinv)


def _tc_gcn_post_body(pm_ref, hp_ref, dinv_ref, b_ref, out_ref):
    pm = jnp.concatenate([pm_ref[0], pm_ref[1]], axis=1)
    hp = jnp.concatenate([hp_ref[0], hp_ref[1]], axis=1)
    out_ref[...] = jnp.maximum(dinv_ref[...] * (pm + hp) + b_ref[...], 0.0)


def _tc_gcn_post_res_body(pm_ref, hp_ref, dinv_ref, b_ref, xp_ref, out_ref):
    pm = jnp.concatenate([pm_ref[0], pm_ref[1]], axis=1)
    hp = jnp.concatenate([hp_ref[0], hp_ref[1]], axis=1)
    out_ref[...] = jnp.maximum(
        dinv_ref[...] * (pm + hp) + b_ref[...] + xp_ref[...], 0.0)


def _tc_gcn_post(pm, hp, dinv, b, xprev=None):
    in_specs = [
        pl.BlockSpec((NC, RB, 64), lambda i: (0, i, 0)),
        pl.BlockSpec((NC, RB, 64), lambda i: (0, i, 0)),
        pl.BlockSpec((RB, 1), lambda i: (i, 0)),
        pl.BlockSpec((1, HID), lambda i: (0, 0)),
    ]
    args = [pm, hp, dinv, b]
    body = _tc_gcn_post_body
    if xprev is not None:
        in_specs.append(pl.BlockSpec((RB, HID), lambda i: (i, 0)))
        args.append(xprev)
        body = _tc_gcn_post_res_body
    return pl.pallas_call(
        body,
        grid=(NP // RB,),
        in_specs=in_specs,
        out_specs=pl.BlockSpec((RB, HID), lambda i: (i, 0)),
        out_shape=jax.ShapeDtypeStruct((NP, HID), F32),
    )(*args)


def _tc_head_pre_body(x_ref, wa_ref, wb_ref, a_ref, b_ref):
    a = _dot(x_ref[...], wa_ref[...])
    b = _dot(x_ref[...], wb_ref[...])
    a_ref[0] = a[:, :64]
    a_ref[1] = a[:, 64:]
    b_ref[0] = b[:, :64]
    b_ref[1] = b[:, 64:]


def _tc_head_pre(xc, w1a, w1b):
    return pl.pallas_call(
        _tc_head_pre_body,
        grid=(NP // RB,),
        in_specs=[
            pl.BlockSpec((RB, HID), lambda i: (i, 0)),
            pl.BlockSpec((HID, HID), lambda i: (0, 0)),
            pl.BlockSpec((HID, HID), lambda i: (0, 0)),
        ],
        out_specs=[
            pl.BlockSpec((NC, RB, 64), lambda i: (0, i, 0)),
            pl.BlockSpec((NC, RB, 64), lambda i: (0, i, 0)),
        ],
        out_shape=[
            jax.ShapeDtypeStruct((NC, NP, 64), F32),
            jax.ShapeDtypeStruct((NC, NP, 64), F32),
        ],
    )(xc, w1a, w1b)


def _tc_head_main_body(as_ref, bs_ref, b1_ref, w2_ref, b2_ref, w3_ref, b3_ref,
                       out_ref):
    a = jnp.concatenate([as_ref[0], as_ref[1]], axis=1)
    b = jnp.concatenate([bs_ref[0], bs_ref[1]], axis=1)
    er = jnp.maximum(a + b + b1_ref[...], 0.0)
    t = jnp.maximum(_dot(er, w2_ref[...]) + b2_ref[...] + er, 0.0)
    out_ref[...] = _dot(t, w3_ref[...]) + b3_ref[...]


def _tc_head_main(as_g, bs_g, b1, w2, b2, w3, b3):
    return pl.pallas_call(
        _tc_head_main_body,
        grid=(E // EB,),
        in_specs=[
            pl.BlockSpec((NC, EB, 64), lambda i: (0, i, 0)),
            pl.BlockSpec((NC, EB, 64), lambda i: (0, i, 0)),
            pl.BlockSpec((1, HID), lambda i: (0, 0)),
            pl.BlockSpec((HID, HID), lambda i: (0, 0)),
            pl.BlockSpec((1, HID), lambda i: (0, 0)),
            pl.BlockSpec((HID, 1), lambda i: (0, 0)),
            pl.BlockSpec((1, 1), lambda i: (0, 0)),
        ],
        out_specs=pl.BlockSpec((EB, 1), lambda i: (i, 0)),
        out_shape=jax.ShapeDtypeStruct((E, 1), F32),
    )(as_g, bs_g, b1, w2, b2, w3, b3)


def kernel(x, edge_attr, enn_w1, enn_b1, enn_w2, enn_b2, root_w, nn_bias,
           conv_ws, conv_bs, mlp_ws, mlp_bs, edge_index):
    src = edge_index[0]
    dst = edge_index[1]

    xp = jnp.pad(x, ((0, NP - N), (0, INP - INC)))
    w2k = enn_w2.reshape(8, INC, EMB)
    b2k = enn_b2.reshape(INC, EMB)
    w2all = jnp.concatenate([w2k, b2k[None]], axis=0)
    w2all = jnp.transpose(w2all, (1, 0, 2)).reshape(INC, 9 * EMB)
    w2all = jnp.pad(w2all, ((0, INP - INC), (0, 0)))
    root_w16 = jnp.pad(root_w, ((0, INP - INC), (0, 0)))
    b1_2d = enn_b1.reshape(1, 8)
    nn_bias_2d = nn_bias.reshape(1, EMB)
    zeros_mw = jnp.zeros((NP, MW), F32)
    zeros_64 = jnp.zeros((NP, 64), F32)
    src_es = src.reshape(NC * NS, ES_CHUNKS, K)
    dst_es = dst.reshape(NC * NS, ES_CHUNKS, K)
    src_fs = src.reshape(NS, FS_G, FS_GC, K)
    dst_fs = dst.reshape(NS, FS_G, FS_GC, K)

    xs = _sc_gather_x(xp, src_es)
    msgs = _tc_msgs(edge_attr, xs, enn_w1, b1_2d, w2all)
    aggdeg = _sc_scatter_msgs(msgs, dst_es, zeros_mw)
    xc, dinv = _tc_x1(aggdeg, xp, root_w16, nn_bias_2d)

    for l in range(NCONV):
        hp = _tc_gcn_pre(xc, conv_ws[l], dinv)
        pm = _sc_gcn_agg(hp, src_fs, dst_fs, zeros_64)
        b2d = conv_bs[l].reshape(1, HID)
        xc = _tc_gcn_post(pm, hp, dinv, b2d, xc if l > 0 else None)

    w1a = mlp_ws[0][:HID]
    w1b = mlp_ws[0][HID:]
    a_t, b_t = _tc_head_pre(xc, w1a, w1b)
    as_g, bs_g = _sc_head_gather(a_t, b_t, src_fs, dst_fs)
    out = _tc_head_main(as_g, bs_g, mlp_bs[0].reshape(1, HID),
                        mlp_ws[1], mlp_bs[1].reshape(1, HID),
                        mlp_ws[2], mlp_bs[2].reshape(1, 1))
    return out


# speedup vs baseline: 7.2497x; 1.2444x over previous
import functools

import jax
import jax.numpy as jnp
from jax import lax
from jax.experimental import pallas as pl
from jax.experimental.pallas import tpu as pltpu
from jax.experimental.pallas import tpu_sc as plsc

N = 10000
NP = 10240
E = 320000
INC = 13
INP = 16
EMB = 32
HID = 128
NCONV = 8
MW = 48

NC = 2
NS = 16
K = 80

ES_PER_W = E // (NC * NS)
ES_CHUNKS = ES_PER_W // K
ES_G = 25
ES_GC = 5
FS_PER_S = E // NS
FS_G = 25
FS_GC = 10
ROWS_PER_S = NP // NS

RB = 1024
EB = 2000

F32 = jnp.float32


def _mesh():
    return plsc.VectorSubcoreMesh(core_axis_name="c", subcore_axis_name="s")


@functools.partial(
    pl.kernel,
    compiler_params=pltpu.CompilerParams(use_tc_tiling_on_sc=False),
    out_type=jax.ShapeDtypeStruct((E, INP), F32),
    mesh=_mesh(),
    scratch_types=[
        pltpu.VMEM_SHARED((NP, INP), F32),
        pltpu.VMEM((ES_CHUNKS, K), jnp.int32),
        pltpu.VMEM((K, INP), F32),
        pltpu.VMEM((K, INP), F32),
        pltpu.SemaphoreType.DMA,
        pltpu.SemaphoreType.DMA,
        pltpu.SemaphoreType.DMA,
        pltpu.SemaphoreType.DMA,
    ],
)
def _sc_gather_x(xp_hbm, srci_hbm, out_hbm, table_sp, idx_v, buf0, buf1,
                 gs0, gs1, ws0, ws1):
    c = lax.axis_index("c")
    s = lax.axis_index("s")
    wid = c * NS + s
    rs = pl.ds(s * ROWS_PER_S, ROWS_PER_S)
    pltpu.sync_copy(xp_hbm.at[rs], table_sp.at[rs])
    pltpu.sync_copy(srci_hbm.at[wid], idx_v)
    plsc.subcore_barrier()
    base = wid * ES_PER_W
    bufs, gsems, wsems = (buf0, buf1), (gs0, gs1), (ws0, ws1)

    def group(g, carry):
        gd = {0: pltpu.async_copy(
            table_sp.at[idx_v.at[g * ES_GC]], bufs[0], gsems[0])}
        wd = {}
        for jj in range(ES_GC):
            b = jj % 2
            gd[jj].wait()
            wd[jj] = pltpu.async_copy(
                bufs[b], out_hbm.at[pl.ds(base + (g * ES_GC + jj) * K, K)],
                wsems[b])
            if jj + 1 < ES_GC:
                if jj >= 1:
                    wd[jj - 1].wait()
                gd[jj + 1] = pltpu.async_copy(
                    table_sp.at[idx_v.at[g * ES_GC + jj + 1]], bufs[1 - b],
                    gsems[1 - b])
        wd[ES_GC - 2].wait()
        wd[ES_GC - 1].wait()
        return carry

    lax.fori_loop(0, ES_G, group, 0)


@functools.partial(
    pl.kernel,
    compiler_params=pltpu.CompilerParams(use_tc_tiling_on_sc=False),
    out_type=jax.ShapeDtypeStruct((NC, NP, MW), F32),
    mesh=_mesh(),
    scratch_types=[
        pltpu.VMEM_SHARED((NP, MW), F32),
        pltpu.VMEM((ES_CHUNKS, K), jnp.int32),
        pltpu.VMEM((K, MW), F32),
        pltpu.VMEM((K, MW), F32),
        pltpu.SemaphoreType.DMA,
        pltpu.SemaphoreType.DMA,
        pltpu.SemaphoreType.DMA,
        pltpu.SemaphoreType.DMA,
    ],
)
def _sc_scatter_msgs(msgs_hbm, dsti_hbm, zeros_hbm, out_hbm, accum_sp, idx_v,
                     buf0, buf1, gs0, gs1, ss0, ss1):
    c = lax.axis_index("c")
    s = lax.axis_index("s")
    wid = c * NS + s
    rs = pl.ds(s * ROWS_PER_S, ROWS_PER_S)
    pltpu.sync_copy(zeros_hbm.at[rs], accum_sp.at[rs])
    pltpu.sync_copy(dsti_hbm.at[wid], idx_v)
    plsc.subcore_barrier()
    base = wid * ES_PER_W
    bufs, gsems, ssems = (buf0, buf1), (gs0, gs1), (ss0, ss1)

    def group(g, carry):
        gd = {0: pltpu.async_copy(
            msgs_hbm.at[pl.ds((base + g * ES_GC * K), K)], bufs[0], gsems[0])}
        sd = {}
        for jj in range(ES_GC):
            b = jj % 2
            gd[jj].wait()
            sd[jj] = pltpu.async_copy(
                bufs[b], accum_sp.at[idx_v.at[g * ES_GC + jj]], ssems[b],
                add=True)
            if jj + 1 < ES_GC:
                if jj >= 1:
                    sd[jj - 1].wait()
                gd[jj + 1] = pltpu.async_copy(
                    msgs_hbm.at[pl.ds(base + (g * ES_GC + jj + 1) * K, K)],
                    bufs[1 - b], gsems[1 - b])
        sd[ES_GC - 2].wait()
        sd[ES_GC - 1].wait()
        return carry

    lax.fori_loop(0, ES_G, group, 0)
    plsc.subcore_barrier()
    pltpu.sync_copy(accum_sp.at[rs], out_hbm.at[c, rs])


@functools.partial(
    pl.kernel,
    compiler_params=pltpu.CompilerParams(use_tc_tiling_on_sc=False),
    out_type=jax.ShapeDtypeStruct((NC, NP, 64), F32),
    mesh=_mesh(),
    scratch_types=[
        pltpu.VMEM_SHARED((NP, 64), F32),
        pltpu.VMEM_SHARED((NP, 64), F32),
        pltpu.VMEM((FS_GC, K), jnp.int32),
        pltpu.VMEM((FS_GC, K), jnp.int32),
        pltpu.VMEM((K, 64), F32),
        pltpu.VMEM((K, 64), F32),
        pltpu.SemaphoreType.DMA,
        pltpu.SemaphoreType.DMA,
        pltpu.SemaphoreType.DMA,
        pltpu.SemaphoreType.DMA,
    ],
)
def _sc_gcn_agg(hp_hbm, srci_hbm, dsti_hbm, zeros_hbm, out_hbm, table_sp,
                accum_sp, src_blk, dst_blk, buf0, buf1, gs0, gs1, ss0, ss1):
    c = lax.axis_index("c")
    s = lax.axis_index("s")
    rs = pl.ds(s * ROWS_PER_S, ROWS_PER_S)
    pltpu.sync_copy(hp_hbm.at[c, rs], table_sp.at[rs])
    pltpu.sync_copy(zeros_hbm.at[rs], accum_sp.at[rs])
    plsc.subcore_barrier()
    bufs, gsems, ssems = (buf0, buf1), (gs0, gs1), (ss0, ss1)

    def group(g, carry):
        pltpu.sync_copy(srci_hbm.at[s, g], src_blk)
        pltpu.sync_copy(dsti_hbm.at[s, g], dst_blk)
        gd = {0: pltpu.async_copy(
            table_sp.at[src_blk.at[0]], bufs[0], gsems[0])}
        sd = {}
        for jj in range(FS_GC):
            b = jj % 2
            gd[jj].wait()
            sd[jj] = pltpu.async_copy(
                bufs[b], accum_sp.at[dst_blk.at[jj]], ssems[b], add=True)
            if jj + 1 < FS_GC:
                if jj >= 1:
                    sd[jj - 1].wait()
                gd[jj + 1] = pltpu.async_copy(
                    table_sp.at[src_blk.at[jj + 1]], bufs[1 - b],
                    gsems[1 - b])
        sd[FS_GC - 2].wait()
        sd[FS_GC - 1].wait()
        return carry

    lax.fori_loop(0, FS_G, group, 0)
    plsc.subcore_barrier()
    pltpu.sync_copy(accum_sp.at[rs], out_hbm.at[c, rs])


@functools.partial(
    pl.kernel,
    compiler_params=pltpu.CompilerParams(use_tc_tiling_on_sc=False),
    out_type=[
        jax.ShapeDtypeStruct((NC, E, 64), F32),
        jax.ShapeDtypeStruct((NC, E, 64), F32),
    ],
    mesh=_mesh(),
    scratch_types=[
        pltpu.VMEM_SHARED((NP, 64), F32),
        pltpu.VMEM_SHARED((NP, 64), F32),
        pltpu.VMEM((FS_GC, K), jnp.int32),
        pltpu.VMEM((FS_GC, K), jnp.int32),
        pltpu.VMEM((K, 64), F32),
        pltpu.VMEM((K, 64), F32),
        pltpu.VMEM((K, 64), F32),
        pltpu.VMEM((K, 64), F32),
        pltpu.SemaphoreType.DMA,
        pltpu.SemaphoreType.DMA,
        pltpu.SemaphoreType.DMA,
        pltpu.SemaphoreType.DMA,
        pltpu.SemaphoreType.DMA,
        pltpu.SemaphoreType.DMA,
        pltpu.SemaphoreType.DMA,
        pltpu.SemaphoreType.DMA,
    ],
)
def _sc_head_gather(a_hbm, b_hbm, srci_hbm, dsti_hbm, as_hbm, bs_hbm, ta_sp,
                    tb_sp, src_blk, dst_blk, ba0, ba1, bb0, bb1,
                    ga0, ga1, gb0, gb1, wa0, wa1, wb0, wb1):
    c = lax.axis_index("c")
    s = lax.axis_index("s")
    rs = pl.ds(s * ROWS_PER_S, ROWS_PER_S)
    pltpu.sync_copy(a_hbm.at[c, rs], ta_sp.at[rs])
    pltpu.sync_copy(b_hbm.at[c, rs], tb_sp.at[rs])
    plsc.subcore_barrier()
    base = s * FS_PER_S
    bas, bbs = (ba0, ba1), (bb0, bb1)
    gas, gbs = (ga0, ga1), (gb0, gb1)
    was, wbs = (wa0, wa1), (wb0, wb1)

    def group(g, carry):
        pltpu.sync_copy(srci_hbm.at[s, g], src_blk)
        pltpu.sync_copy(dsti_hbm.at[s, g], dst_blk)
        gda = {0: pltpu.async_copy(ta_sp.at[src_blk.at[0]], bas[0], gas[0])}
        gdb = {0: pltpu.async_copy(tb_sp.at[dst_blk.at[0]], bbs[0], gbs[0])}
        wda, wdb = {}, {}
        for jj in range(FS_GC):
            b = jj % 2
            gda[jj].wait()
            gdb[jj].wait()
            off = base + (g * FS_GC + jj) * K
            wda[jj] = pltpu.async_copy(
                bas[b], as_hbm.at[c, pl.ds(off, K)], was[b])
            wdb[jj] = pltpu.async_copy(
                bbs[b], bs_hbm.at[c, pl.ds(off, K)], wbs[b])
            if jj + 1 < FS_GC:
                if jj >= 1:
                    wda[jj - 1].wait()
                    wdb[jj - 1].wait()
                gda[jj + 1] = pltpu.async_copy(
                    ta_sp.at[src_blk.at[jj + 1]], bas[1 - b], gas[1 - b])
                gdb[jj + 1] = pltpu.async_copy(
                    tb_sp.at[dst_blk.at[jj + 1]], bbs[1 - b], gbs[1 - b])
        for jj in (FS_GC - 2, FS_GC - 1):
            wda[jj].wait()
            wdb[jj].wait()
        return carry

    lax.fori_loop(0, FS_G, group, 0)


def _dot(a, b):
    return jnp.dot(a, b, preferred_element_type=F32)


def _tc_msgs_body(ea_ref, xs_ref, w1_ref, b1_ref, w2all_ref, rep_ref,
                  crow_ref, summat_ref, out_ref):
    h = jnp.maximum(_dot(ea_ref[...], w1_ref[...]) + b1_ref[...], 0.0)
    z = _dot(xs_ref[...], w2all_ref[...])
    hr = _dot(h, rep_ref[...]) + crow_ref[...]
    m = _dot(hr * z, summat_ref[...])
    out_ref[:, 0:EMB] = m
    out_ref[:, EMB:EMB + 1] = jnp.ones((EB, 1), F32)
    out_ref[:, EMB + 1:MW] = jnp.zeros((EB, MW - EMB - 1), F32)


def _tc_msgs(edge_attr, xs, enn_w1, enn_b1, w2all, rep, crow, summat):
    return pl.pallas_call(
        _tc_msgs_body,
        grid=(E // EB,),
        in_specs=[
            pl.BlockSpec((EB, 3), lambda i: (i, 0)),
            pl.BlockSpec((EB, INP), lambda i: (i, 0)),
            pl.BlockSpec((3, 8), lambda i: (0, 0)),
            pl.BlockSpec((1, 8), lambda i: (0, 0)),
            pl.BlockSpec((INP, 9 * EMB), lambda i: (0, 0)),
            pl.BlockSpec((8, 9 * EMB), lambda i: (0, 0)),
            pl.BlockSpec((1, 9 * EMB), lambda i: (0, 0)),
            pl.BlockSpec((9 * EMB, EMB), lambda i: (0, 0)),
        ],
        out_specs=pl.BlockSpec((EB, MW), lambda i: (i, 0)),
        out_shape=jax.ShapeDtypeStruct((E, MW), F32),
    )(edge_attr, xs, enn_w1, enn_b1, w2all, rep, crow, summat)


def _tc_x1_body(ad_ref, xp_ref, rw_ref, nb_ref, x1_ref, dinv_ref):
    ad = ad_ref[0] + ad_ref[1]
    agg = ad[:, :EMB]
    deg = ad[:, EMB:EMB + 1] + 1.0
    dinv_ref[...] = lax.rsqrt(deg)
    x1_ref[...] = jnp.maximum(
        agg + _dot(xp_ref[...], rw_ref[...]) + nb_ref[...], 0.0)


def _tc_x1(aggdeg, xp, root_w, nn_bias):
    return pl.pallas_call(
        _tc_x1_body,
        grid=(NP // RB,),
        in_specs=[
            pl.BlockSpec((NC, RB, MW), lambda i: (0, i, 0)),
            pl.BlockSpec((RB, INP), lambda i: (i, 0)),
            pl.BlockSpec((INP, EMB), lambda i: (0, 0)),
            pl.BlockSpec((1, EMB), lambda i: (0, 0)),
        ],
        out_specs=[
            pl.BlockSpec((RB, EMB), lambda i: (i, 0)),
            pl.BlockSpec((RB, 1), lambda i: (i, 0)),
        ],
        out_shape=[
            jax.ShapeDtypeStruct((NP, EMB), F32),
            jax.ShapeDtypeStruct((NP, 1), F32),
        ],
    )(aggdeg, xp, root_w, nn_bias)


def _tc_gcn_pre_body(x_ref, w_ref, dinv_ref, hp_ref):
    hp = dinv_ref[...] * _dot(x_ref[...], w_ref[...])
    hp_ref[0] = hp[:, :64]
    hp_ref[1] = hp[:, 64:]


def _tc_gcn_pre(xc, w, dinv):
    cin = xc.shape[1]
    return pl.pallas_call(
        _tc_gcn_pre_body,
        grid=(NP // RB,),
        in_specs=[
            pl.BlockSpec((RB, cin), lambda i: (i, 0)),
            pl.BlockSpec((cin, HID), lambda i: (0, 0)),
            pl.BlockSpec((RB, 1), lambda i: (i, 0)),
        ],
        out_specs=pl.BlockSpec((NC, RB, 64), lambda i: (0, i, 0)),
        out_shape=jax.ShapeDtypeStruct((NC, NP, 64), F32),
    )(xc, w, dinv)


def _tc_gcn_post_body(pm_ref, hp_ref, dinv_ref, b_ref, out_ref):
    pm = jnp.concatenate([pm_ref[0], pm_ref[1]], axis=1)
    hp = jnp.concatenate([hp_ref[0], hp_ref[1]], axis=1)
    out_ref[...] = jnp.maximum(dinv_ref[...] * (pm + hp) + b_ref[...], 0.0)


def _tc_gcn_post_res_body(pm_ref, hp_ref, dinv_ref, b_ref, xp_ref, out_ref):
    pm = jnp.concatenate([pm_ref[0], pm_ref[1]], axis=1)
    hp = jnp.concatenate([hp_ref[0], hp_ref[1]], axis=1)
    out_ref[...] = jnp.maximum(
        dinv_ref[...] * (pm + hp) + b_ref[...] + xp_ref[...], 0.0)


def _tc_gcn_post(pm, hp, dinv, b, xprev=None):
    in_specs = [
        pl.BlockSpec((NC, RB, 64), lambda i: (0, i, 0)),
        pl.BlockSpec((NC, RB, 64), lambda i: (0, i, 0)),
        pl.BlockSpec((RB, 1), lambda i: (i, 0)),
        pl.BlockSpec((1, HID), lambda i: (0, 0)),
    ]
    args = [pm, hp, dinv, b]
    body = _tc_gcn_post_body
    if xprev is not None:
        in_specs.append(pl.BlockSpec((RB, HID), lambda i: (i, 0)))
        args.append(xprev)
        body = _tc_gcn_post_res_body
    return pl.pallas_call(
        body,
        grid=(NP // RB,),
        in_specs=in_specs,
        out_specs=pl.BlockSpec((RB, HID), lambda i: (i, 0)),
        out_shape=jax.ShapeDtypeStruct((NP, HID), F32),
    )(*args)


def _tc_head_pre_body(x_ref, wa_ref, wb_ref, a_ref, b_ref):
    a = _dot(x_ref[...], wa_ref[...])
    b = _dot(x_ref[...], wb_ref[...])
    a_ref[0] = a[:, :64]
    a_ref[1] = a[:, 64:]
    b_ref[0] = b[:, :64]
    b_ref[1] = b[:, 64:]


def _tc_head_pre(xc, w1a, w1b):
    return pl.pallas_call(
        _tc_head_pre_body,
        grid=(NP // RB,),
        in_specs=[
            pl.BlockSpec((RB, HID), lambda i: (i, 0)),
            pl.BlockSpec((HID, HID), lambda i: (0, 0)),
            pl.BlockSpec((HID, HID), lambda i: (0, 0)),
        ],
        out_specs=[
            pl.BlockSpec((NC, RB, 64), lambda i: (0, i, 0)),
            pl.BlockSpec((NC, RB, 64), lambda i: (0, i, 0)),
        ],
        out_shape=[
            jax.ShapeDtypeStruct((NC, NP, 64), F32),
            jax.ShapeDtypeStruct((NC, NP, 64), F32),
        ],
    )(xc, w1a, w1b)


def _tc_head_main_body(as_ref, bs_ref, b1_ref, w2_ref, b2_ref, w3_ref, b3_ref,
                       out_ref):
    a = jnp.concatenate([as_ref[0], as_ref[1]], axis=1)
    b = jnp.concatenate([bs_ref[0], bs_ref[1]], axis=1)
    er = jnp.maximum(a + b + b1_ref[...], 0.0)
    t = jnp.maximum(_dot(er, w2_ref[...]) + b2_ref[...] + er, 0.0)
    out_ref[...] = _dot(t, w3_ref[...]) + b3_ref[...]


def _tc_head_main(as_g, bs_g, b1, w2, b2, w3, b3):
    return pl.pallas_call(
        _tc_head_main_body,
        grid=(E // EB,),
        in_specs=[
            pl.BlockSpec((NC, EB, 64), lambda i: (0, i, 0)),
            pl.BlockSpec((NC, EB, 64), lambda i: (0, i, 0)),
            pl.BlockSpec((1, HID), lambda i: (0, 0)),
            pl.BlockSpec((HID, HID), lambda i: (0, 0)),
            pl.BlockSpec((1, HID), lambda i: (0, 0)),
            pl.BlockSpec((HID, 1), lambda i: (0, 0)),
            pl.BlockSpec((1, 1), lambda i: (0, 0)),
        ],
        out_specs=pl.BlockSpec((EB, 1), lambda i: (i, 0)),
        out_shape=jax.ShapeDtypeStruct((E, 1), F32),
    )(as_g, bs_g, b1, w2, b2, w3, b3)


def kernel(x, edge_attr, enn_w1, enn_b1, enn_w2, enn_b2, root_w, nn_bias,
           conv_ws, conv_bs, mlp_ws, mlp_bs, edge_index):
    src = edge_index[0]
    dst = edge_index[1]

    xp = jnp.pad(x, ((0, NP - N), (0, INP - INC)))
    w2k = enn_w2.reshape(8, INC, EMB)
    b2k = enn_b2.reshape(INC, EMB)
    w2all = jnp.concatenate([w2k, b2k[None]], axis=0)
    w2all = jnp.transpose(w2all, (1, 0, 2)).reshape(INC, 9 * EMB)
    w2all = jnp.pad(w2all, ((0, INP - INC), (0, 0)))
    root_w16 = jnp.pad(root_w, ((0, INP - INC), (0, 0)))
    b1_2d = enn_b1.reshape(1, 8)
    nn_bias_2d = nn_bias.reshape(1, EMB)
    rep = jnp.repeat(jnp.eye(8, 9, dtype=F32), EMB, axis=1)
    crow = jnp.repeat(jnp.eye(1, 9, 8, dtype=F32), EMB, axis=1)
    summat = jnp.tile(jnp.eye(EMB, dtype=F32), (9, 1))
    zeros_mw = jnp.zeros((NP, MW), F32)
    zeros_64 = jnp.zeros((NP, 64), F32)
    src_es = src.reshape(NC * NS, ES_CHUNKS, K)
    dst_es = dst.reshape(NC * NS, ES_CHUNKS, K)
    src_fs = src.reshape(NS, FS_G, FS_GC, K)
    dst_fs = dst.reshape(NS, FS_G, FS_GC, K)

    xs = _sc_gather_x(xp, src_es)
    msgs = _tc_msgs(edge_attr, xs, enn_w1, b1_2d, w2all,
                    rep, crow, summat)
    aggdeg = _sc_scatter_msgs(msgs, dst_es, zeros_mw)
    xc, dinv = _tc_x1(aggdeg, xp, root_w16, nn_bias_2d)

    for l in range(NCONV):
        hp = _tc_gcn_pre(xc, conv_ws[l], dinv)
        pm = _sc_gcn_agg(hp, src_fs, dst_fs, zeros_64)
        b2d = conv_bs[l].reshape(1, HID)
        xc = _tc_gcn_post(pm, hp, dinv, b2d, xc if l > 0 else None)

    w1a = mlp_ws[0][:HID]
    w1b = mlp_ws[0][HID:]
    a_t, b_t = _tc_head_pre(xc, w1a, w1b)
    as_g, bs_g = _sc_head_gather(a_t, b_t, src_fs, dst_fs)
    out = _tc_head_main(as_g, bs_g, mlp_bs[0].reshape(1, HID),
                        mlp_ws[1], mlp_bs[1].reshape(1, HID),
                        mlp_ws[2], mlp_bs[2].reshape(1, 1))
    return out


# speedup vs baseline: 7.4598x; 1.0290x over previous
import functools

import jax
import jax.numpy as jnp
from jax import lax
from jax.experimental import pallas as pl
from jax.experimental.pallas import tpu as pltpu
from jax.experimental.pallas import tpu_sc as plsc

N = 10000
NP = 10240
E = 320000
INC = 13
INP = 16
EMB = 32
HID = 128
NCONV = 8
MW = 48

NC = 2
NS = 16
K = 80
KE = 80

ES_PER_W = E // (NC * NS)
ES_CHUNKS = ES_PER_W // KE
ES_G = 25
ES_GC = 5
FS_PER_S = E // NS
FS_G = 25
FS_GC = 10
ROWS_PER_S = NP // NS

RB = 1024
EB = 2000

F32 = jnp.float32


def _mesh():
    return plsc.VectorSubcoreMesh(core_axis_name="c", subcore_axis_name="s")


@functools.partial(
    pl.kernel,
    compiler_params=pltpu.CompilerParams(use_tc_tiling_on_sc=False),
    out_type=jax.ShapeDtypeStruct((E, INP), F32),
    mesh=_mesh(),
    scratch_types=[
        pltpu.VMEM_SHARED((NP, INP), F32),
        pltpu.VMEM((ES_CHUNKS, KE), jnp.int32),
        pltpu.VMEM((KE, INP), F32),
        pltpu.VMEM((KE, INP), F32),
        pltpu.SemaphoreType.DMA,
        pltpu.SemaphoreType.DMA,
        pltpu.SemaphoreType.DMA,
        pltpu.SemaphoreType.DMA,
    ],
)
def _sc_gather_x(xp_hbm, srci_hbm, out_hbm, table_sp, idx_v, buf0, buf1,
                 gs0, gs1, ws0, ws1):
    c = lax.axis_index("c")
    s = lax.axis_index("s")
    wid = c * NS + s
    rs = pl.ds(s * ROWS_PER_S, ROWS_PER_S)
    pltpu.sync_copy(xp_hbm.at[rs], table_sp.at[rs])
    pltpu.sync_copy(srci_hbm.at[wid], idx_v)
    plsc.subcore_barrier()
    base = wid * ES_PER_W
    bufs, gsems, wsems = (buf0, buf1), (gs0, gs1), (ws0, ws1)

    def group(g, carry):
        gd = {0: pltpu.async_copy(
            table_sp.at[idx_v.at[g * ES_GC]], bufs[0], gsems[0])}
        wd = {}
        for jj in range(ES_GC):
            b = jj % 2
            gd[jj].wait()
            wd[jj] = pltpu.async_copy(
                bufs[b], out_hbm.at[pl.ds(base + (g * ES_GC + jj) * KE, KE)],
                wsems[b])
            if jj + 1 < ES_GC:
                if jj >= 1:
                    wd[jj - 1].wait()
                gd[jj + 1] = pltpu.async_copy(
                    table_sp.at[idx_v.at[g * ES_GC + jj + 1]], bufs[1 - b],
                    gsems[1 - b])
        wd[ES_GC - 2].wait()
        wd[ES_GC - 1].wait()
        return carry

    lax.fori_loop(0, ES_G, group, 0)


@functools.partial(
    pl.kernel,
    compiler_params=pltpu.CompilerParams(use_tc_tiling_on_sc=False),
    out_type=jax.ShapeDtypeStruct((NC, NP, MW), F32),
    mesh=_mesh(),
    scratch_types=[
        pltpu.VMEM_SHARED((NP, MW), F32),
        pltpu.VMEM((ES_CHUNKS, KE), jnp.int32),
        pltpu.VMEM((KE, MW), F32),
        pltpu.VMEM((KE, MW), F32),
        pltpu.VMEM((KE, MW), F32),
        pltpu.SemaphoreType.DMA,
        pltpu.SemaphoreType.DMA,
        pltpu.SemaphoreType.DMA,
        pltpu.SemaphoreType.DMA,
        pltpu.SemaphoreType.DMA,
        pltpu.SemaphoreType.DMA,
    ],
)
def _sc_scatter_msgs(msgs_hbm, dsti_hbm, zeros_hbm, out_hbm, accum_sp, idx_v,
                     buf0, buf1, buf2, gs0, gs1, gs2, ss0, ss1, ss2):
    c = lax.axis_index("c")
    s = lax.axis_index("s")
    wid = c * NS + s
    rs = pl.ds(s * ROWS_PER_S, ROWS_PER_S)
    pltpu.sync_copy(zeros_hbm.at[rs], accum_sp.at[rs])
    pltpu.sync_copy(dsti_hbm.at[wid], idx_v)
    plsc.subcore_barrier()
    base = wid * ES_PER_W
    bufs = (buf0, buf1, buf2)
    gsems = (gs0, gs1, gs2)
    ssems = (ss0, ss1, ss2)

    def group(g, carry):
        gd = {0: pltpu.async_copy(
            msgs_hbm.at[pl.ds(base + g * ES_GC * KE, KE)], bufs[0], gsems[0]),
              1: pltpu.async_copy(
            msgs_hbm.at[pl.ds(base + (g * ES_GC + 1) * KE, KE)], bufs[1],
            gsems[1])}
        sd = {}
        for jj in range(ES_GC):
            b = jj % 3
            gd[jj].wait()
            sd[jj] = pltpu.async_copy(
                bufs[b], accum_sp.at[idx_v.at[g * ES_GC + jj]], ssems[b],
                add=True)
            if jj + 2 < ES_GC:
                nb = (jj + 2) % 3
                if jj >= 1:
                    sd[jj - 1].wait()
                gd[jj + 2] = pltpu.async_copy(
                    msgs_hbm.at[pl.ds(base + (g * ES_GC + jj + 2) * KE, KE)],
                    bufs[nb], gsems[nb])
        for t in (ES_GC - 3, ES_GC - 2, ES_GC - 1):
            sd[t].wait()
        return carry

    lax.fori_loop(0, ES_G, group, 0)
    plsc.subcore_barrier()
    pltpu.sync_copy(accum_sp.at[rs], out_hbm.at[c, rs])


@functools.partial(
    pl.kernel,
    compiler_params=pltpu.CompilerParams(use_tc_tiling_on_sc=False),
    out_type=jax.ShapeDtypeStruct((NC, NP, 64), F32),
    mesh=_mesh(),
    scratch_types=[
        pltpu.VMEM_SHARED((NP, 64), F32),
        pltpu.VMEM_SHARED((NP, 64), F32),
        pltpu.VMEM((FS_GC, K), jnp.int32),
        pltpu.VMEM((FS_GC, K), jnp.int32),
        pltpu.VMEM((K, 64), F32),
        pltpu.VMEM((K, 64), F32),
        pltpu.VMEM((K, 64), F32),
        pltpu.SemaphoreType.DMA,
        pltpu.SemaphoreType.DMA,
        pltpu.SemaphoreType.DMA,
        pltpu.SemaphoreType.DMA,
        pltpu.SemaphoreType.DMA,
        pltpu.SemaphoreType.DMA,
    ],
)
def _sc_gcn_agg(hp_hbm, srci_hbm, dsti_hbm, zeros_hbm, out_hbm, table_sp,
                accum_sp, src_blk, dst_blk, buf0, buf1, buf2,
                gs0, gs1, gs2, ss0, ss1, ss2):
    c = lax.axis_index("c")
    s = lax.axis_index("s")
    rs = pl.ds(s * ROWS_PER_S, ROWS_PER_S)
    pltpu.sync_copy(hp_hbm.at[c, rs], table_sp.at[rs])
    pltpu.sync_copy(zeros_hbm.at[rs], accum_sp.at[rs])
    plsc.subcore_barrier()
    bufs = (buf0, buf1, buf2)
    gsems = (gs0, gs1, gs2)
    ssems = (ss0, ss1, ss2)

    def group(g, carry):
        pltpu.sync_copy(srci_hbm.at[s, g], src_blk)
        pltpu.sync_copy(dsti_hbm.at[s, g], dst_blk)
        gd = {0: pltpu.async_copy(table_sp.at[src_blk.at[0]], bufs[0],
                                  gsems[0]),
              1: pltpu.async_copy(table_sp.at[src_blk.at[1]], bufs[1],
                                  gsems[1])}
        sd = {}
        for jj in range(FS_GC):
            b = jj % 3
            gd[jj].wait()
            sd[jj] = pltpu.async_copy(
                bufs[b], accum_sp.at[dst_blk.at[jj]], ssems[b], add=True)
            if jj + 2 < FS_GC:
                nb = (jj + 2) % 3
                if jj >= 1:
                    sd[jj - 1].wait()
                gd[jj + 2] = pltpu.async_copy(
                    table_sp.at[src_blk.at[jj + 2]], bufs[nb], gsems[nb])
        for t in (FS_GC - 3, FS_GC - 2, FS_GC - 1):
            sd[t].wait()
        return carry

    lax.fori_loop(0, FS_G, group, 0)
    plsc.subcore_barrier()
    pltpu.sync_copy(accum_sp.at[rs], out_hbm.at[c, rs])


@functools.partial(
    pl.kernel,
    compiler_params=pltpu.CompilerParams(use_tc_tiling_on_sc=False),
    out_type=[
        jax.ShapeDtypeStruct((NC, E, 64), F32),
        jax.ShapeDtypeStruct((NC, E, 64), F32),
    ],
    mesh=_mesh(),
    scratch_types=[
        pltpu.VMEM_SHARED((NP, 64), F32),
        pltpu.VMEM_SHARED((NP, 64), F32),
        pltpu.VMEM((FS_GC, K), jnp.int32),
        pltpu.VMEM((FS_GC, K), jnp.int32),
        pltpu.VMEM((K, 64), F32),
        pltpu.VMEM((K, 64), F32),
        pltpu.VMEM((K, 64), F32),
        pltpu.VMEM((K, 64), F32),
        pltpu.VMEM((K, 64), F32),
        pltpu.VMEM((K, 64), F32),
        pltpu.SemaphoreType.DMA,
        pltpu.SemaphoreType.DMA,
        pltpu.SemaphoreType.DMA,
        pltpu.SemaphoreType.DMA,
        pltpu.SemaphoreType.DMA,
        pltpu.SemaphoreType.DMA,
        pltpu.SemaphoreType.DMA,
        pltpu.SemaphoreType.DMA,
        pltpu.SemaphoreType.DMA,
        pltpu.SemaphoreType.DMA,
        pltpu.SemaphoreType.DMA,
        pltpu.SemaphoreType.DMA,
    ],
)
def _sc_head_gather(a_hbm, b_hbm, srci_hbm, dsti_hbm, as_hbm, bs_hbm, ta_sp,
                    tb_sp, src_blk, dst_blk, ba0, ba1, ba2, bb0, bb1, bb2,
                    ga0, ga1, ga2, gb0, gb1, gb2, wa0, wa1, wa2,
                    wb0, wb1, wb2):
    c = lax.axis_index("c")
    s = lax.axis_index("s")
    rs = pl.ds(s * ROWS_PER_S, ROWS_PER_S)
    pltpu.sync_copy(a_hbm.at[c, rs], ta_sp.at[rs])
    pltpu.sync_copy(b_hbm.at[c, rs], tb_sp.at[rs])
    plsc.subcore_barrier()
    base = s * FS_PER_S
    bas, bbs = (ba0, ba1, ba2), (bb0, bb1, bb2)
    gas, gbs = (ga0, ga1, ga2), (gb0, gb1, gb2)
    was, wbs = (wa0, wa1, wa2), (wb0, wb1, wb2)

    def group(g, carry):
        pltpu.sync_copy(srci_hbm.at[s, g], src_blk)
        pltpu.sync_copy(dsti_hbm.at[s, g], dst_blk)
        gda = {0: pltpu.async_copy(ta_sp.at[src_blk.at[0]], bas[0], gas[0]),
               1: pltpu.async_copy(ta_sp.at[src_blk.at[1]], bas[1], gas[1])}
        gdb = {0: pltpu.async_copy(tb_sp.at[dst_blk.at[0]], bbs[0], gbs[0]),
               1: pltpu.async_copy(tb_sp.at[dst_blk.at[1]], bbs[1], gbs[1])}
        wda, wdb = {}, {}
        for jj in range(FS_GC):
            b = jj % 3
            gda[jj].wait()
            gdb[jj].wait()
            off = base + (g * FS_GC + jj) * K
            wda[jj] = pltpu.async_copy(
                bas[b], as_hbm.at[c, pl.ds(off, K)], was[b])
            wdb[jj] = pltpu.async_copy(
                bbs[b], bs_hbm.at[c, pl.ds(off, K)], wbs[b])
            if jj + 2 < FS_GC:
                nb = (jj + 2) % 3
                if jj >= 1:
                    wda[jj - 1].wait()
                    wdb[jj - 1].wait()
                gda[jj + 2] = pltpu.async_copy(
                    ta_sp.at[src_blk.at[jj + 2]], bas[nb], gas[nb])
                gdb[jj + 2] = pltpu.async_copy(
                    tb_sp.at[dst_blk.at[jj + 2]], bbs[nb], gbs[nb])
        for jj in (FS_GC - 3, FS_GC - 2, FS_GC - 1):
            wda[jj].wait()
            wdb[jj].wait()
        return carry

    lax.fori_loop(0, FS_G, group, 0)


def _dot(a, b):
    return jnp.dot(a, b, preferred_element_type=F32)


def _tc_msgs_body(ea_ref, xs_ref, w1_ref, b1_ref, w2all_ref, rep_ref,
                  crow_ref, summat_ref, out_ref):
    h = jnp.maximum(_dot(ea_ref[...], w1_ref[...]) + b1_ref[...], 0.0)
    z = _dot(xs_ref[...], w2all_ref[...])
    hr = _dot(h, rep_ref[...]) + crow_ref[...]
    m = _dot(hr * z, summat_ref[...])
    out_ref[:, 0:EMB] = m
    out_ref[:, EMB:EMB + 1] = jnp.ones((EB, 1), F32)
    out_ref[:, EMB + 1:MW] = jnp.zeros((EB, MW - EMB - 1), F32)


def _tc_msgs(edge_attr, xs, enn_w1, enn_b1, w2all, rep, crow, summat):
    return pl.pallas_call(
        _tc_msgs_body,
        grid=(E // EB,),
        in_specs=[
            pl.BlockSpec((EB, 3), lambda i: (i, 0)),
            pl.BlockSpec((EB, INP), lambda i: (i, 0)),
            pl.BlockSpec((3, 8), lambda i: (0, 0)),
            pl.BlockSpec((1, 8), lambda i: (0, 0)),
            pl.BlockSpec((INP, 9 * EMB), lambda i: (0, 0)),
            pl.BlockSpec((8, 9 * EMB), lambda i: (0, 0)),
            pl.BlockSpec((1, 9 * EMB), lambda i: (0, 0)),
            pl.BlockSpec((9 * EMB, EMB), lambda i: (0, 0)),
        ],
        out_specs=pl.BlockSpec((EB, MW), lambda i: (i, 0)),
        out_shape=jax.ShapeDtypeStruct((E, MW), F32),
    )(edge_attr, xs, enn_w1, enn_b1, w2all, rep, crow, summat)


def _tc_x1_body(ad_ref, xp_ref, rw_ref, nb_ref, x1_ref, dinv_ref):
    ad = ad_ref[0] + ad_ref[1]
    agg = ad[:, :EMB]
    deg = ad[:, EMB:EMB + 1] + 1.0
    dinv_ref[...] = lax.rsqrt(deg)
    x1_ref[...] = jnp.maximum(
        agg + _dot(xp_ref[...], rw_ref[...]) + nb_ref[...], 0.0)


def _tc_x1(aggdeg, xp, root_w, nn_bias):
    return pl.pallas_call(
        _tc_x1_body,
        grid=(NP // RB,),
        in_specs=[
            pl.BlockSpec((NC, RB, MW), lambda i: (0, i, 0)),
            pl.BlockSpec((RB, INP), lambda i: (i, 0)),
            pl.BlockSpec((INP, EMB), lambda i: (0, 0)),
            pl.BlockSpec((1, EMB), lambda i: (0, 0)),
        ],
        out_specs=[
            pl.BlockSpec((RB, EMB), lambda i: (i, 0)),
            pl.BlockSpec((RB, 1), lambda i: (i, 0)),
        ],
        out_shape=[
            jax.ShapeDtypeStruct((NP, EMB), F32),
            jax.ShapeDtypeStruct((NP, 1), F32),
        ],
    )(aggdeg, xp, root_w, nn_bias)


def _tc_gcn_pre_body(x_ref, w_ref, dinv_ref, hp_ref):
    hp = dinv_ref[...] * _dot(x_ref[...], w_ref[...])
    hp_ref[0] = hp[:, :64]
    hp_ref[1] = hp[:, 64:]


def _tc_gcn_pre(xc, w, dinv):
    cin = xc.shape[1]
    return pl.pallas_call(
        _tc_gcn_pre_body,
        grid=(NP // RB,),
        in_specs=[
            pl.BlockSpec((RB, cin), lambda i: (i, 0)),
            pl.BlockSpec((cin, HID), lambda i: (0, 0)),
            pl.BlockSpec((RB, 1), lambda i: (i, 0)),
        ],
        out_specs=pl.BlockSpec((NC, RB, 64), lambda i: (0, i, 0)),
        out_shape=jax.ShapeDtypeStruct((NC, NP, 64), F32),
    )(xc, w, dinv)


def _tc_gcn_step_body(pm_ref, hp_ref, dinv_ref, b_ref, w_ref, x_ref,
                      hpn_ref):
    pm = jnp.concatenate([pm_ref[0], pm_ref[1]], axis=1)
    hp = jnp.concatenate([hp_ref[0], hp_ref[1]], axis=1)
    x = jnp.maximum(dinv_ref[...] * (pm + hp) + b_ref[...], 0.0)
    hpn = dinv_ref[...] * _dot(x, w_ref[...])
    x_ref[...] = x
    hpn_ref[0] = hpn[:, :64]
    hpn_ref[1] = hpn[:, 64:]


def _tc_gcn_step_res_body(pm_ref, hp_ref, dinv_ref, b_ref, w_ref, xp_ref,
                          x_ref, hpn_ref):
    pm = jnp.concatenate([pm_ref[0], pm_ref[1]], axis=1)
    hp = jnp.concatenate([hp_ref[0], hp_ref[1]], axis=1)
    x = jnp.maximum(dinv_ref[...] * (pm + hp) + b_ref[...] + xp_ref[...], 0.0)
    hpn = dinv_ref[...] * _dot(x, w_ref[...])
    x_ref[...] = x
    hpn_ref[0] = hpn[:, :64]
    hpn_ref[1] = hpn[:, 64:]


def _tc_gcn_step(pm, hp, dinv, b, w_next, xprev=None):
    in_specs = [
        pl.BlockSpec((NC, RB, 64), lambda i: (0, i, 0)),
        pl.BlockSpec((NC, RB, 64), lambda i: (0, i, 0)),
        pl.BlockSpec((RB, 1), lambda i: (i, 0)),
        pl.BlockSpec((1, HID), lambda i: (0, 0)),
        pl.BlockSpec((HID, HID), lambda i: (0, 0)),
    ]
    args = [pm, hp, dinv, b, w_next]
    body = _tc_gcn_step_body
    if xprev is not None:
        in_specs.append(pl.BlockSpec((RB, HID), lambda i: (i, 0)))
        args.append(xprev)
        body = _tc_gcn_step_res_body
    return pl.pallas_call(
        body,
        grid=(NP // RB,),
        in_specs=in_specs,
        out_specs=[
            pl.BlockSpec((RB, HID), lambda i: (i, 0)),
            pl.BlockSpec((NC, RB, 64), lambda i: (0, i, 0)),
        ],
        out_shape=[
            jax.ShapeDtypeStruct((NP, HID), F32),
            jax.ShapeDtypeStruct((NC, NP, 64), F32),
        ],
    )(*args)


def _tc_gcn_post_body(pm_ref, hp_ref, dinv_ref, b_ref, out_ref):
    pm = jnp.concatenate([pm_ref[0], pm_ref[1]], axis=1)
    hp = jnp.concatenate([hp_ref[0], hp_ref[1]], axis=1)
    out_ref[...] = jnp.maximum(dinv_ref[...] * (pm + hp) + b_ref[...], 0.0)


def _tc_gcn_post_res_body(pm_ref, hp_ref, dinv_ref, b_ref, xp_ref, out_ref):
    pm = jnp.concatenate([pm_ref[0], pm_ref[1]], axis=1)
    hp = jnp.concatenate([hp_ref[0], hp_ref[1]], axis=1)
    out_ref[...] = jnp.maximum(
        dinv_ref[...] * (pm + hp) + b_ref[...] + xp_ref[...], 0.0)


def _tc_gcn_post(pm, hp, dinv, b, xprev=None):
    in_specs = [
        pl.BlockSpec((NC, RB, 64), lambda i: (0, i, 0)),
        pl.BlockSpec((NC, RB, 64), lambda i: (0, i, 0)),
        pl.BlockSpec((RB, 1), lambda i: (i, 0)),
        pl.BlockSpec((1, HID), lambda i: (0, 0)),
    ]
    args = [pm, hp, dinv, b]
    body = _tc_gcn_post_body
    if xprev is not None:
        in_specs.append(pl.BlockSpec((RB, HID), lambda i: (i, 0)))
        args.append(xprev)
        body = _tc_gcn_post_res_body
    return pl.pallas_call(
        body,
        grid=(NP // RB,),
        in_specs=in_specs,
        out_specs=pl.BlockSpec((RB, HID), lambda i: (i, 0)),
        out_shape=jax.ShapeDtypeStruct((NP, HID), F32),
    )(*args)


def _tc_head_pre_body(x_ref, wa_ref, wb_ref, a_ref, b_ref):
    a = _dot(x_ref[...], wa_ref[...])
    b = _dot(x_ref[...], wb_ref[...])
    a_ref[0] = a[:, :64]
    a_ref[1] = a[:, 64:]
    b_ref[0] = b[:, :64]
    b_ref[1] = b[:, 64:]


def _tc_head_pre(xc, w1a, w1b):
    return pl.pallas_call(
        _tc_head_pre_body,
        grid=(NP // RB,),
        in_specs=[
            pl.BlockSpec((RB, HID), lambda i: (i, 0)),
            pl.BlockSpec((HID, HID), lambda i: (0, 0)),
            pl.BlockSpec((HID, HID), lambda i: (0, 0)),
        ],
        out_specs=[
            pl.BlockSpec((NC, RB, 64), lambda i: (0, i, 0)),
            pl.BlockSpec((NC, RB, 64), lambda i: (0, i, 0)),
        ],
        out_shape=[
            jax.ShapeDtypeStruct((NC, NP, 64), F32),
            jax.ShapeDtypeStruct((NC, NP, 64), F32),
        ],
    )(xc, w1a, w1b)


def _tc_head_main_body(as_ref, bs_ref, b1_ref, w2_ref, b2_ref, w3_ref, b3_ref,
                       out_ref):
    a = jnp.concatenate([as_ref[0], as_ref[1]], axis=1)
    b = jnp.concatenate([bs_ref[0], bs_ref[1]], axis=1)
    er = jnp.maximum(a + b + b1_ref[...], 0.0)
    t = jnp.maximum(_dot(er, w2_ref[...]) + b2_ref[...] + er, 0.0)
    out_ref[...] = _dot(t, w3_ref[...]) + b3_ref[...]


def _tc_head_main(as_g, bs_g, b1, w2, b2, w3, b3):
    return pl.pallas_call(
        _tc_head_main_body,
        grid=(E // EB,),
        in_specs=[
            pl.BlockSpec((NC, EB, 64), lambda i: (0, i, 0)),
            pl.BlockSpec((NC, EB, 64), lambda i: (0, i, 0)),
            pl.BlockSpec((1, HID), lambda i: (0, 0)),
            pl.BlockSpec((HID, HID), lambda i: (0, 0)),
            pl.BlockSpec((1, HID), lambda i: (0, 0)),
            pl.BlockSpec((HID, 1), lambda i: (0, 0)),
            pl.BlockSpec((1, 1), lambda i: (0, 0)),
        ],
        out_specs=pl.BlockSpec((EB, 1), lambda i: (i, 0)),
        out_shape=jax.ShapeDtypeStruct((E, 1), F32),
    )(as_g, bs_g, b1, w2, b2, w3, b3)


def kernel(x, edge_attr, enn_w1, enn_b1, enn_w2, enn_b2, root_w, nn_bias,
           conv_ws, conv_bs, mlp_ws, mlp_bs, edge_index):
    src = edge_index[0]
    dst = edge_index[1]

    xp = jnp.pad(x, ((0, NP - N), (0, INP - INC)))
    w2k = enn_w2.reshape(8, INC, EMB)
    b2k = enn_b2.reshape(INC, EMB)
    w2all = jnp.concatenate([w2k, b2k[None]], axis=0)
    w2all = jnp.transpose(w2all, (1, 0, 2)).reshape(INC, 9 * EMB)
    w2all = jnp.pad(w2all, ((0, INP - INC), (0, 0)))
    root_w16 = jnp.pad(root_w, ((0, INP - INC), (0, 0)))
    b1_2d = enn_b1.reshape(1, 8)
    nn_bias_2d = nn_bias.reshape(1, EMB)
    rep = jnp.repeat(jnp.eye(8, 9, dtype=F32), EMB, axis=1)
    crow = jnp.repeat(jnp.eye(1, 9, 8, dtype=F32), EMB, axis=1)
    summat = jnp.tile(jnp.eye(EMB, dtype=F32), (9, 1))
    zeros_mw = jnp.zeros((NP, MW), F32)
    zeros_64 = jnp.zeros((NP, 64), F32)
    src_es = src.reshape(NC * NS, ES_CHUNKS, KE)
    dst_es = dst.reshape(NC * NS, ES_CHUNKS, KE)
    src_fs = src.reshape(NS, FS_G, FS_GC, K)
    dst_fs = dst.reshape(NS, FS_G, FS_GC, K)

    xs = _sc_gather_x(xp, src_es)
    msgs = _tc_msgs(edge_attr, xs, enn_w1, b1_2d, w2all,
                    rep, crow, summat)
    aggdeg = _sc_scatter_msgs(msgs, dst_es, zeros_mw)
    xc, dinv = _tc_x1(aggdeg, xp, root_w16, nn_bias_2d)

    hp = _tc_gcn_pre(xc, conv_ws[0], dinv)
    for l in range(NCONV):
        pm = _sc_gcn_agg(hp, src_fs, dst_fs, zeros_64)
        b2d = conv_bs[l].reshape(1, HID)
        xprev = xc if l > 0 else None
        if l < NCONV - 1:
            xc, hp = _tc_gcn_step(pm, hp, dinv, b2d, conv_ws[l + 1], xprev)
        else:
            xc = _tc_gcn_post(pm, hp, dinv, b2d, xprev)

    w1a = mlp_ws[0][:HID]
    w1b = mlp_ws[0][HID:]
    a_t, b_t = _tc_head_pre(xc, w1a, w1b)
    as_g, bs_g = _sc_head_gather(a_t, b_t, src_fs, dst_fs)
    out = _tc_head_main(as_g, bs_g, mlp_bs[0].reshape(1, HID),
                        mlp_ws[1], mlp_bs[1].reshape(1, HID),
                        mlp_ws[2], mlp_bs[2].reshape(1, 1))
    return out
